# baseline (speedup 1.0000x reference)
import functools

import jax
import jax.numpy as jnp
from jax import lax
from jax.experimental import pallas as pl
from jax.experimental.pallas import tpu as pltpu
from jax.experimental.pallas import tpu_sc as plsc

L = 200
D = 64
LANES = 16
NV = D // LANES
CHUNKS = ((0, 128), (128, 72))
NBUF = 4
UNROLL = 8


def _make_sc_pool(B):
    info = plsc.get_sparse_core_info()
    NC, NS = info.num_cores, info.num_subcores
    NW = NC * NS
    rows_per_w = B // NW
    mesh = plsc.VectorSubcoreMesh(core_axis_name="c", subcore_axis_name="s")

    @functools.partial(
        pl.kernel,
        out_type=jax.ShapeDtypeStruct((B, 4 * D), jnp.float32),
        mesh=mesh,
        compiler_params=pltpu.CompilerParams(use_tc_tiling_on_sc=False),
        scratch_types=[
            pltpu.VMEM((rows_per_w, L), jnp.int32),
            pltpu.VMEM((NBUF, L, D), jnp.float32),
            pltpu.VMEM((rows_per_w, 4 * D), jnp.float32),
            [pltpu.SemaphoreType.DMA] * NBUF,
        ],
    )
    def sc_pool(x_hbm, table_hbm, out_hbm, xs_v, rows_v, acc_v, sems):
        wid = lax.axis_index("s") * NC + lax.axis_index("c")
        base = wid * rows_per_w
        pltpu.sync_copy(x_hbm.at[pl.ds(base, rows_per_w)], xs_v)

        def fire(i, buf):
            for off, n in CHUNKS:
                pltpu.async_copy(
                    table_hbm.at[xs_v.at[i, pl.ds(off, n)]],
                    rows_v.at[buf].at[pl.ds(off, n)],
                    sems[buf],
                )

        def drain(i, buf):
            for off, n in CHUNKS:
                pltpu.make_async_copy(
                    table_hbm.at[xs_v.at[i, pl.ds(off, n)]],
                    rows_v.at[buf].at[pl.ds(off, n)],
                    sems[buf],
                ).wait()

        def reduce_store(i, buf):
            rbuf = rows_v.at[buf]

            def red_body(k, st):
                sums, sqs, mxs, mns = st
                r0 = k * UNROLL
                for u in range(UNROLL):
                    ns, nq, nx, nn = [], [], [], []
                    for c in range(NV):
                        v = rbuf[r0 + u, pl.ds(c * LANES, LANES)]
                        ns.append(sums[c] + v)
                        nq.append(sqs[c] + v * v)
                        nx.append(jnp.maximum(mxs[c], v))
                        nn.append(jnp.minimum(mns[c], v))
                    sums, sqs, mxs, mns = tuple(ns), tuple(nq), tuple(nx), tuple(nn)
                return sums, sqs, mxs, mns

            z = tuple(jnp.zeros((LANES,), jnp.float32) for _ in range(NV))
            neg = tuple(jnp.full((LANES,), -jnp.inf, jnp.float32) for _ in range(NV))
            pos = tuple(jnp.full((LANES,), jnp.inf, jnp.float32) for _ in range(NV))
            sums, sqs, mxs, mns = lax.fori_loop(
                0, L // UNROLL, red_body, (z, z, neg, pos)
            )

            for c in range(NV):
                acc_v[i, pl.ds(c * LANES, LANES)] = sums[c]
                acc_v[i, pl.ds(D + c * LANES, LANES)] = mxs[c]
                acc_v[i, pl.ds(2 * D + c * LANES, LANES)] = mns[c]
                acc_v[i, pl.ds(3 * D + c * LANES, LANES)] = sqs[c]

        for b in range(NBUF):
            fire(b, b)

        def outer_body(g, carry):
            for b in range(NBUF):
                i = g * NBUF + b
                drain(i, b)
                reduce_store(i, b)

                @pl.when(i + NBUF < rows_per_w)
                def _():
                    fire(i + NBUF, b)

            return carry

        lax.fori_loop(0, rows_per_w // NBUF, outer_body, 0)
        pltpu.sync_copy(acc_v, out_hbm.at[pl.ds(base, rows_per_w)])

    return sc_pool


RETILE_CB = 2048


def _tc_retile(tableT):
    Dd, V = tableT.shape
    CB = RETILE_CB
    H = CB // 2
    nblk = pl.cdiv(V, CB)

    def body(in_ref, o_ref):
        t = in_ref[...]
        tt = jax.lax.dot_general(
            t,
            jnp.eye(Dd, dtype=jnp.float32),
            (((0,), (0,)), ((), ())),
            preferred_element_type=jnp.float32,
            precision=jax.lax.Precision.HIGHEST,
        )
        pair = jnp.concatenate([tt[:H], tt[H:]], axis=1)
        o_ref[...] = pair.reshape(H * 2 * Dd)

    return pl.pallas_call(
        body,
        grid=(nblk,),
        in_specs=[pl.BlockSpec((Dd, CB), lambda i: (0, i))],
        out_specs=pl.BlockSpec((CB * Dd,), lambda i: (i,)),
        out_shape=jax.ShapeDtypeStruct((nblk * CB * Dd,), jnp.float32),
    )(tableT)


def _remap_idx(x):
    return (x & ~(RETILE_CB - 1)) | (
        ((x & (RETILE_CB // 2 - 1)) << 1) | ((x >> 10) & 1)
    )


def _tc_tail(acc, W, b):
    B = acc.shape[0]
    bs = 512

    def body(acc_ref, w_ref, b_ref, o_ref):
        a = acc_ref[...]
        s = a[:, 0 * D:1 * D]
        mx = a[:, 1 * D:2 * D]
        mn = a[:, 2 * D:3 * D]
        sq = a[:, 3 * D:4 * D]
        mean = s * (1.0 / L)
        var = (sq - s * mean) * (1.0 / (L - 1))
        std = jnp.sqrt(jnp.maximum(var, 0.0))
        feat = jnp.concatenate([mean, mx, mn, std], axis=1)
        o_ref[...] = (
            jnp.dot(feat, w_ref[...], preferred_element_type=jnp.float32)
            + b_ref[...]
        )

    return pl.pallas_call(
        body,
        grid=(B // bs,),
        in_specs=[
            pl.BlockSpec((bs, 4 * D), lambda i: (i, 0)),
            pl.BlockSpec((4 * D, 2), lambda i: (0, 0)),
            pl.BlockSpec((1, 2), lambda i: (0, 0)),
        ],
        out_specs=pl.BlockSpec((bs, 2), lambda i: (i, 0)),
        out_shape=jax.ShapeDtypeStruct((B, 2), jnp.float32),
    )(acc, W, b.reshape(1, 2))


def kernel(x, table, W, b):
    B = x.shape[0]
    V = table.shape[0]
    x = _remap_idx(x.astype(jnp.int32))
    tbl_lin = _tc_retile(table.T)
    tbl2 = tbl_lin.reshape(tbl_lin.shape[0] // D, D)
    acc = _make_sc_pool(B)(x, tbl2)
    return _tc_tail(acc, W, b)


# speedup vs baseline: 1.4098x; 1.4098x over previous
import functools

import jax
import jax.numpy as jnp
from jax import lax
from jax.experimental import pallas as pl
from jax.experimental.pallas import tpu as pltpu
from jax.experimental.pallas import tpu_sc as plsc

L = 200
D = 64
LANES = 16
NV = D // LANES
CHUNKS = ((0, 128), (128, 72))
NBUF = 4
UNROLL = 8


def _make_sc_pool(B):
    info = plsc.get_sparse_core_info()
    NC, NS = info.num_cores, info.num_subcores
    NW = NC * NS
    rows_per_w = B // NW
    mesh = plsc.VectorSubcoreMesh(core_axis_name="c", subcore_axis_name="s")

    @functools.partial(
        pl.kernel,
        out_type=jax.ShapeDtypeStruct((B, 4 * D), jnp.float32),
        mesh=mesh,
        compiler_params=pltpu.CompilerParams(use_tc_tiling_on_sc=False),
        scratch_types=[
            pltpu.VMEM((rows_per_w, L), jnp.int32),
            pltpu.VMEM((NBUF, L, D), jnp.float32),
            pltpu.VMEM((rows_per_w, 4 * D), jnp.float32),
            [pltpu.SemaphoreType.DMA] * NBUF,
        ],
    )
    def sc_pool(x_hbm, table_hbm, out_hbm, xs_v, rows_v, acc_v, sems):
        wid = lax.axis_index("s") * NC + lax.axis_index("c")
        base = wid * rows_per_w
        pltpu.sync_copy(x_hbm.at[pl.ds(base, rows_per_w)], xs_v)

        def fire(i, buf):
            for off, n in CHUNKS:
                pltpu.async_copy(
                    table_hbm.at[xs_v.at[i, pl.ds(off, n)]],
                    rows_v.at[buf].at[pl.ds(off, n)],
                    sems[buf],
                )

        def drain(i, buf):
            for off, n in CHUNKS:
                pltpu.make_async_copy(
                    table_hbm.at[xs_v.at[i, pl.ds(off, n)]],
                    rows_v.at[buf].at[pl.ds(off, n)],
                    sems[buf],
                ).wait()

        def reduce_store(i, buf):
            rbuf = rows_v.at[buf]

            def red_body(k, st):
                sums, sqs, mxs, mns = st
                r0 = k * UNROLL
                for u in range(UNROLL):
                    ns, nq, nx, nn = [], [], [], []
                    for c in range(NV):
                        v = rbuf[r0 + u, pl.ds(c * LANES, LANES)]
                        ns.append(sums[c] + v)
                        nq.append(sqs[c] + v * v)
                        nx.append(jnp.maximum(mxs[c], v))
                        nn.append(jnp.minimum(mns[c], v))
                    sums, sqs, mxs, mns = tuple(ns), tuple(nq), tuple(nx), tuple(nn)
                return sums, sqs, mxs, mns

            z = tuple(jnp.zeros((LANES,), jnp.float32) for _ in range(NV))
            neg = tuple(jnp.full((LANES,), -jnp.inf, jnp.float32) for _ in range(NV))
            pos = tuple(jnp.full((LANES,), jnp.inf, jnp.float32) for _ in range(NV))
            sums, sqs, mxs, mns = lax.fori_loop(
                0, L // UNROLL, red_body, (z, z, neg, pos)
            )

            for c in range(NV):
                acc_v[i, pl.ds(c * LANES, LANES)] = sums[c]
                acc_v[i, pl.ds(D + c * LANES, LANES)] = mxs[c]
                acc_v[i, pl.ds(2 * D + c * LANES, LANES)] = mns[c]
                acc_v[i, pl.ds(3 * D + c * LANES, LANES)] = sqs[c]

        for b in range(NBUF):
            fire(b, b)

        def outer_body(g, carry):
            for b in range(NBUF):
                i = g * NBUF + b
                drain(i, b)
                reduce_store(i, b)

                @pl.when(i + NBUF < rows_per_w)
                def _():
                    fire(i + NBUF, b)

            return carry

        lax.fori_loop(0, rows_per_w // NBUF, outer_body, 0)
        pltpu.sync_copy(acc_v, out_hbm.at[pl.ds(base, rows_per_w)])

    return sc_pool


RETILE_CB = 2048


def _tc_retile(tableT):
    Dd, V = tableT.shape
    CB = RETILE_CB
    H = CB // 2
    nblk = pl.cdiv(V, CB)

    def body(in_ref, o_ref):
        t = in_ref[...]
        stacked = jnp.concatenate([t[:, :H], t[:, H:]], axis=0)
        pair = jnp.swapaxes(stacked, 0, 1)
        o_ref[...] = pair.reshape(H * 2 * Dd)

    return pl.pallas_call(
        body,
        grid=(nblk,),
        in_specs=[pl.BlockSpec((Dd, CB), lambda i: (0, i))],
        out_specs=pl.BlockSpec((CB * Dd,), lambda i: (i,)),
        out_shape=jax.ShapeDtypeStruct((nblk * CB * Dd,), jnp.float32),
    )(tableT)


def _remap_idx(x):
    return (x & ~(RETILE_CB - 1)) | (
        ((x & (RETILE_CB // 2 - 1)) << 1) | ((x >> 10) & 1)
    )


def _tc_tail(acc, W, b):
    B = acc.shape[0]
    bs = 512

    def body(acc_ref, w_ref, b_ref, o_ref):
        a = acc_ref[...]
        s = a[:, 0 * D:1 * D]
        mx = a[:, 1 * D:2 * D]
        mn = a[:, 2 * D:3 * D]
        sq = a[:, 3 * D:4 * D]
        mean = s * (1.0 / L)
        var = (sq - s * mean) * (1.0 / (L - 1))
        std = jnp.sqrt(jnp.maximum(var, 0.0))
        feat = jnp.concatenate([mean, mx, mn, std], axis=1)
        o_ref[...] = (
            jnp.dot(feat, w_ref[...], preferred_element_type=jnp.float32)
            + b_ref[...]
        )

    return pl.pallas_call(
        body,
        grid=(B // bs,),
        in_specs=[
            pl.BlockSpec((bs, 4 * D), lambda i: (i, 0)),
            pl.BlockSpec((4 * D, 2), lambda i: (0, 0)),
            pl.BlockSpec((1, 2), lambda i: (0, 0)),
        ],
        out_specs=pl.BlockSpec((bs, 2), lambda i: (i, 0)),
        out_shape=jax.ShapeDtypeStruct((B, 2), jnp.float32),
    )(acc, W, b.reshape(1, 2))


def kernel(x, table, W, b):
    B = x.shape[0]
    V = table.shape[0]
    x = _remap_idx(x.astype(jnp.int32))
    tbl_lin = _tc_retile(table.T)
    tbl2 = tbl_lin.reshape(tbl_lin.shape[0] // D, D)
    acc = _make_sc_pool(B)(x, tbl2)
    return _tc_tail(acc, W, b)


# speedup vs baseline: 2.1765x; 1.5438x over previous
import functools

import jax
import jax.numpy as jnp
from jax import lax
from jax.experimental import pallas as pl
from jax.experimental.pallas import tpu as pltpu
from jax.experimental.pallas import tpu_sc as plsc

L = 200
D = 64
LANES = 16
NV = D // LANES
CHUNKS = ((0, 128), (128, 72))
NBUF = 4
UNROLL = 8


def _make_sc_pool(B):
    info = plsc.get_sparse_core_info()
    NC, NS = info.num_cores, info.num_subcores
    NW = NC * NS
    rows_per_w = B // NW
    mesh = plsc.VectorSubcoreMesh(core_axis_name="c", subcore_axis_name="s")

    @functools.partial(
        pl.kernel,
        out_type=jax.ShapeDtypeStruct((B, 4 * D), jnp.float32),
        mesh=mesh,
        compiler_params=pltpu.CompilerParams(use_tc_tiling_on_sc=False),
        scratch_types=[
            pltpu.VMEM((rows_per_w, L), jnp.int32),
            pltpu.VMEM((NBUF, L, D), jnp.float32),
            pltpu.VMEM((rows_per_w, 4 * D), jnp.float32),
            [pltpu.SemaphoreType.DMA] * NBUF,
        ],
    )
    def sc_pool(x_hbm, table_hbm, out_hbm, xs_v, rows_v, acc_v, sems):
        wid = lax.axis_index("s") * NC + lax.axis_index("c")
        base = wid * rows_per_w
        pltpu.sync_copy(x_hbm.at[pl.ds(base, rows_per_w)], xs_v)

        def fire(i, buf):
            for off, n in CHUNKS:
                pltpu.async_copy(
                    table_hbm.at[xs_v.at[i, pl.ds(off, n)]],
                    rows_v.at[buf].at[pl.ds(off, n)],
                    sems[buf],
                )

        def drain(i, buf):
            for off, n in CHUNKS:
                pltpu.make_async_copy(
                    table_hbm.at[xs_v.at[i, pl.ds(off, n)]],
                    rows_v.at[buf].at[pl.ds(off, n)],
                    sems[buf],
                ).wait()

        def reduce_store(i, buf):
            rbuf = rows_v.at[buf]

            def red_body(k, st):
                sums, sqs, mxs, mns = st
                r0 = k * UNROLL
                for u in range(UNROLL):
                    ns, nq, nx, nn = [], [], [], []
                    for c in range(NV):
                        v = rbuf[r0 + u, pl.ds(c * LANES, LANES)]
                        ns.append(sums[c] + v)
                        nq.append(sqs[c] + v * v)
                        nx.append(jnp.maximum(mxs[c], v))
                        nn.append(jnp.minimum(mns[c], v))
                    sums, sqs, mxs, mns = tuple(ns), tuple(nq), tuple(nx), tuple(nn)
                return sums, sqs, mxs, mns

            z = tuple(jnp.zeros((LANES,), jnp.float32) for _ in range(NV))
            neg = tuple(jnp.full((LANES,), -jnp.inf, jnp.float32) for _ in range(NV))
            pos = tuple(jnp.full((LANES,), jnp.inf, jnp.float32) for _ in range(NV))
            sums, sqs, mxs, mns = lax.fori_loop(
                0, L // UNROLL, red_body, (z, z, neg, pos)
            )

            for c in range(NV):
                acc_v[i, pl.ds(c * LANES, LANES)] = sums[c]
                acc_v[i, pl.ds(D + c * LANES, LANES)] = mxs[c]
                acc_v[i, pl.ds(2 * D + c * LANES, LANES)] = mns[c]
                acc_v[i, pl.ds(3 * D + c * LANES, LANES)] = sqs[c]

        for b in range(NBUF):
            fire(b, b)

        def outer_body(g, carry):
            for b in range(NBUF):
                i = g * NBUF + b
                drain(i, b)
                reduce_store(i, b)

                @pl.when(i + NBUF < rows_per_w)
                def _():
                    fire(i + NBUF, b)

            return carry

        lax.fori_loop(0, rows_per_w // NBUF, outer_body, 0)
        pltpu.sync_copy(acc_v, out_hbm.at[pl.ds(base, rows_per_w)])

    return sc_pool


RETILE_CB = 8192


def _tc_retile(tableT):
    Dd, V = tableT.shape
    CB = RETILE_CB
    H = CB // 2
    nblk = pl.cdiv(V, CB)

    def body(in_ref, o_ref):
        t = in_ref[...]
        stacked = jnp.concatenate([t[:, :H], t[:, H:]], axis=0)
        pair = jnp.swapaxes(stacked, 0, 1)
        o_ref[...] = pair.reshape(H * 2 * Dd)

    return pl.pallas_call(
        body,
        grid=(nblk,),
        in_specs=[pl.BlockSpec((Dd, CB), lambda i: (0, i))],
        out_specs=pl.BlockSpec((CB * Dd,), lambda i: (i,)),
        out_shape=jax.ShapeDtypeStruct((nblk * CB * Dd,), jnp.float32),
    )(tableT)


def _remap_idx(x):
    h = RETILE_CB // 2
    hbit = h.bit_length() - 1
    return (x & ~(RETILE_CB - 1)) | (((x & (h - 1)) << 1) | ((x >> hbit) & 1))


def _tc_tail(acc, W, b):
    B = acc.shape[0]
    bs = 512

    def body(acc_ref, w_ref, b_ref, o_ref):
        a = acc_ref[...]
        s = a[:, 0 * D:1 * D]
        mx = a[:, 1 * D:2 * D]
        mn = a[:, 2 * D:3 * D]
        sq = a[:, 3 * D:4 * D]
        mean = s * (1.0 / L)
        var = (sq - s * mean) * (1.0 / (L - 1))
        std = jnp.sqrt(jnp.maximum(var, 0.0))
        feat = jnp.concatenate([mean, mx, mn, std], axis=1)
        o_ref[...] = (
            jnp.dot(feat, w_ref[...], preferred_element_type=jnp.float32)
            + b_ref[...]
        )

    return pl.pallas_call(
        body,
        grid=(B // bs,),
        in_specs=[
            pl.BlockSpec((bs, 4 * D), lambda i: (i, 0)),
            pl.BlockSpec((4 * D, 2), lambda i: (0, 0)),
            pl.BlockSpec((1, 2), lambda i: (0, 0)),
        ],
        out_specs=pl.BlockSpec((bs, 2), lambda i: (i, 0)),
        out_shape=jax.ShapeDtypeStruct((B, 2), jnp.float32),
    )(acc, W, b.reshape(1, 2))


def kernel(x, table, W, b):
    B = x.shape[0]
    V = table.shape[0]
    x = _remap_idx(x.astype(jnp.int32))
    tbl_lin = _tc_retile(table.T)
    tbl2 = tbl_lin.reshape(tbl_lin.shape[0] // D, D)
    acc = _make_sc_pool(B)(x, tbl2)
    return _tc_tail(acc, W, b)


# speedup vs baseline: 2.3684x; 1.0882x over previous
import functools

import jax
import jax.numpy as jnp
from jax import lax
from jax.experimental import pallas as pl
from jax.experimental.pallas import tpu as pltpu
from jax.experimental.pallas import tpu_sc as plsc

L = 200
D = 64
LANES = 16
NV = D // LANES
CHUNKS = ((0, 128), (128, 72))
NBUF = 4
UNROLL = 8


def _make_sc_pool(B):
    info = plsc.get_sparse_core_info()
    NC, NS = info.num_cores, info.num_subcores
    NW = NC * NS
    rows_per_w = B // NW
    mesh = plsc.VectorSubcoreMesh(core_axis_name="c", subcore_axis_name="s")

    @functools.partial(
        pl.kernel,
        out_type=jax.ShapeDtypeStruct((B, 4 * D), jnp.float32),
        mesh=mesh,
        compiler_params=pltpu.CompilerParams(use_tc_tiling_on_sc=False),
        scratch_types=[
            pltpu.VMEM((rows_per_w, L), jnp.int32),
            pltpu.VMEM((NBUF, L, D), jnp.float32),
            pltpu.VMEM((rows_per_w, 4 * D), jnp.float32),
            [pltpu.SemaphoreType.DMA] * NBUF,
        ],
    )
    def sc_pool(x_hbm, table_hbm, out_hbm, xs_v, rows_v, acc_v, sems):
        wid = lax.axis_index("s") * NC + lax.axis_index("c")
        base = wid * rows_per_w
        pltpu.sync_copy(x_hbm.at[pl.ds(base, rows_per_w)], xs_v)

        def fire(i, buf):
            for off, n in CHUNKS:
                pltpu.async_copy(
                    table_hbm.at[xs_v.at[i, pl.ds(off, n)]],
                    rows_v.at[buf].at[pl.ds(off, n)],
                    sems[buf],
                )

        def drain(i, buf):
            for off, n in CHUNKS:
                pltpu.make_async_copy(
                    table_hbm.at[xs_v.at[i, pl.ds(off, n)]],
                    rows_v.at[buf].at[pl.ds(off, n)],
                    sems[buf],
                ).wait()

        def reduce_store(i, buf):
            rbuf = rows_v.at[buf]

            def red_body(k, st):
                sums, sqs, mxs, mns = st
                r0 = k * UNROLL
                for u in range(UNROLL):
                    ns, nq, nx, nn = [], [], [], []
                    for c in range(NV):
                        v = rbuf[r0 + u, pl.ds(c * LANES, LANES)]
                        ns.append(sums[c] + v)
                        nq.append(sqs[c] + v * v)
                        nx.append(jnp.maximum(mxs[c], v))
                        nn.append(jnp.minimum(mns[c], v))
                    sums, sqs, mxs, mns = tuple(ns), tuple(nq), tuple(nx), tuple(nn)
                return sums, sqs, mxs, mns

            z = tuple(jnp.zeros((LANES,), jnp.float32) for _ in range(NV))
            neg = tuple(jnp.full((LANES,), -jnp.inf, jnp.float32) for _ in range(NV))
            pos = tuple(jnp.full((LANES,), jnp.inf, jnp.float32) for _ in range(NV))
            sums, sqs, mxs, mns = lax.fori_loop(
                0, L // UNROLL, red_body, (z, z, neg, pos)
            )

            for c in range(NV):
                acc_v[i, pl.ds(c * LANES, LANES)] = sums[c]
                acc_v[i, pl.ds(D + c * LANES, LANES)] = mxs[c]
                acc_v[i, pl.ds(2 * D + c * LANES, LANES)] = mns[c]
                acc_v[i, pl.ds(3 * D + c * LANES, LANES)] = sqs[c]

        for b in range(NBUF):
            fire(b, b)

        def outer_body(g, carry):
            for b in range(NBUF):
                i = g * NBUF + b
                drain(i, b)
                reduce_store(i, b)

                @pl.when(i + NBUF < rows_per_w)
                def _():
                    fire(i + NBUF, b)

            return carry

        lax.fori_loop(0, rows_per_w // NBUF, outer_body, 0)
        pltpu.sync_copy(acc_v, out_hbm.at[pl.ds(base, rows_per_w)])

    return sc_pool


RETILE_CB = 16384


def _tc_retile(tableT):
    Dd, V = tableT.shape
    CB = RETILE_CB
    H = CB // 2
    nblk = pl.cdiv(V, CB)

    def body(in_ref, o_ref):
        t = in_ref[...]
        stacked = jnp.concatenate([t[:, :H], t[:, H:]], axis=0)
        pair = jnp.swapaxes(stacked, 0, 1)
        o_ref[...] = pair.reshape(H * 2 * Dd)

    return pl.pallas_call(
        body,
        grid=(nblk,),
        in_specs=[pl.BlockSpec((Dd, CB), lambda i: (0, i))],
        out_specs=pl.BlockSpec((CB * Dd,), lambda i: (i,)),
        out_shape=jax.ShapeDtypeStruct((nblk * CB * Dd,), jnp.float32),
    )(tableT)


def _remap_idx(x):
    h = RETILE_CB // 2
    hbit = h.bit_length() - 1
    return (x & ~(RETILE_CB - 1)) | (((x & (h - 1)) << 1) | ((x >> hbit) & 1))


def _tc_tail(acc, W, b):
    B = acc.shape[0]
    bs = 512

    def body(acc_ref, w_ref, b_ref, o_ref):
        a = acc_ref[...]
        s = a[:, 0 * D:1 * D]
        mx = a[:, 1 * D:2 * D]
        mn = a[:, 2 * D:3 * D]
        sq = a[:, 3 * D:4 * D]
        mean = s * (1.0 / L)
        var = (sq - s * mean) * (1.0 / (L - 1))
        std = jnp.sqrt(jnp.maximum(var, 0.0))
        feat = jnp.concatenate([mean, mx, mn, std], axis=1)
        o_ref[...] = (
            jnp.dot(feat, w_ref[...], preferred_element_type=jnp.float32)
            + b_ref[...]
        )

    return pl.pallas_call(
        body,
        grid=(B // bs,),
        in_specs=[
            pl.BlockSpec((bs, 4 * D), lambda i: (i, 0)),
            pl.BlockSpec((4 * D, 2), lambda i: (0, 0)),
            pl.BlockSpec((1, 2), lambda i: (0, 0)),
        ],
        out_specs=pl.BlockSpec((bs, 2), lambda i: (i, 0)),
        out_shape=jax.ShapeDtypeStruct((B, 2), jnp.float32),
    )(acc, W, b.reshape(1, 2))


def kernel(x, table, W, b):
    B = x.shape[0]
    V = table.shape[0]
    x = _remap_idx(x.astype(jnp.int32))
    tbl_lin = _tc_retile(table.T)
    tbl2 = tbl_lin.reshape(tbl_lin.shape[0] // D, D)
    acc = _make_sc_pool(B)(x, tbl2)
    return _tc_tail(acc, W, b)


# speedup vs baseline: 2.4067x; 1.0162x over previous
import functools

import jax
import jax.numpy as jnp
from jax import lax
from jax.experimental import pallas as pl
from jax.experimental.pallas import tpu as pltpu
from jax.experimental.pallas import tpu_sc as plsc

L = 200
D = 64
LANES = 16
NV = D // LANES
CHUNKS = ((0, 128), (128, 72))
NBUF = 4
UNROLL = 8


def _make_sc_pool(B):
    info = plsc.get_sparse_core_info()
    NC, NS = info.num_cores, info.num_subcores
    NW = NC * NS
    rows_per_w = B // NW
    mesh = plsc.VectorSubcoreMesh(core_axis_name="c", subcore_axis_name="s")

    @functools.partial(
        pl.kernel,
        out_type=jax.ShapeDtypeStruct((B, 4 * D), jnp.float32),
        mesh=mesh,
        compiler_params=pltpu.CompilerParams(use_tc_tiling_on_sc=False),
        scratch_types=[
            pltpu.VMEM((rows_per_w, L), jnp.int32),
            pltpu.VMEM((NBUF, L, D), jnp.float32),
            pltpu.VMEM((rows_per_w, 4 * D), jnp.float32),
            [pltpu.SemaphoreType.DMA] * NBUF,
        ],
    )
    def sc_pool(x_hbm, table_hbm, out_hbm, xs_v, rows_v, acc_v, sems):
        wid = lax.axis_index("s") * NC + lax.axis_index("c")
        base = wid * rows_per_w
        pltpu.sync_copy(x_hbm.at[pl.ds(base, rows_per_w)], xs_v)

        def fire(i, buf):
            for off, n in CHUNKS:
                pltpu.async_copy(
                    table_hbm.at[xs_v.at[i, pl.ds(off, n)]],
                    rows_v.at[buf].at[pl.ds(off, n)],
                    sems[buf],
                )

        def drain(i, buf):
            for off, n in CHUNKS:
                pltpu.make_async_copy(
                    table_hbm.at[xs_v.at[i, pl.ds(off, n)]],
                    rows_v.at[buf].at[pl.ds(off, n)],
                    sems[buf],
                ).wait()

        def reduce_store(i, buf):
            rbuf = rows_v.at[buf]

            def red_body(k, st):
                sums, sqs, mxs, mns = st
                r0 = k * UNROLL
                for u in range(UNROLL):
                    ns, nq, nx, nn = [], [], [], []
                    for c in range(NV):
                        v = rbuf[r0 + u, pl.ds(c * LANES, LANES)]
                        ns.append(sums[c] + v)
                        nq.append(sqs[c] + v * v)
                        nx.append(jnp.maximum(mxs[c], v))
                        nn.append(jnp.minimum(mns[c], v))
                    sums, sqs, mxs, mns = tuple(ns), tuple(nq), tuple(nx), tuple(nn)
                return sums, sqs, mxs, mns

            z = tuple(jnp.zeros((LANES,), jnp.float32) for _ in range(NV))
            neg = tuple(jnp.full((LANES,), -jnp.inf, jnp.float32) for _ in range(NV))
            pos = tuple(jnp.full((LANES,), jnp.inf, jnp.float32) for _ in range(NV))
            sums, sqs, mxs, mns = lax.fori_loop(
                0, L // UNROLL, red_body, (z, z, neg, pos)
            )

            for c in range(NV):
                acc_v[i, pl.ds(c * LANES, LANES)] = sums[c]
                acc_v[i, pl.ds(D + c * LANES, LANES)] = mxs[c]
                acc_v[i, pl.ds(2 * D + c * LANES, LANES)] = mns[c]
                acc_v[i, pl.ds(3 * D + c * LANES, LANES)] = sqs[c]

        for b in range(NBUF):
            fire(b, b)

        def outer_body(g, carry):
            for b in range(NBUF):
                i = g * NBUF + b
                drain(i, b)
                reduce_store(i, b)

                @pl.when(i + NBUF < rows_per_w)
                def _():
                    fire(i + NBUF, b)

            return carry

        lax.fori_loop(0, rows_per_w // NBUF, outer_body, 0)
        pltpu.sync_copy(acc_v, out_hbm.at[pl.ds(base, rows_per_w)])

    return sc_pool


RETILE_CB = 32768


def _tc_retile(tableT):
    Dd, V = tableT.shape
    CB = RETILE_CB
    H = CB // 2
    nblk = pl.cdiv(V, CB)

    def body(in_ref, o_ref):
        t = in_ref[...]
        stacked = jnp.concatenate([t[:, :H], t[:, H:]], axis=0)
        pair = jnp.swapaxes(stacked, 0, 1)
        o_ref[...] = pair.reshape(H * 2 * Dd)

    return pl.pallas_call(
        body,
        grid=(nblk,),
        in_specs=[pl.BlockSpec((Dd, CB), lambda i: (0, i))],
        out_specs=pl.BlockSpec((CB * Dd,), lambda i: (i,)),
        out_shape=jax.ShapeDtypeStruct((nblk * CB * Dd,), jnp.float32),
    )(tableT)


def _remap_idx(x):
    h = RETILE_CB // 2
    hbit = h.bit_length() - 1
    return (x & ~(RETILE_CB - 1)) | (((x & (h - 1)) << 1) | ((x >> hbit) & 1))


def _tc_tail(acc, W, b):
    B = acc.shape[0]
    bs = 512

    def body(acc_ref, w_ref, b_ref, o_ref):
        a = acc_ref[...]
        s = a[:, 0 * D:1 * D]
        mx = a[:, 1 * D:2 * D]
        mn = a[:, 2 * D:3 * D]
        sq = a[:, 3 * D:4 * D]
        mean = s * (1.0 / L)
        var = (sq - s * mean) * (1.0 / (L - 1))
        std = jnp.sqrt(jnp.maximum(var, 0.0))
        feat = jnp.concatenate([mean, mx, mn, std], axis=1)
        o_ref[...] = (
            jnp.dot(feat, w_ref[...], preferred_element_type=jnp.float32)
            + b_ref[...]
        )

    return pl.pallas_call(
        body,
        grid=(B // bs,),
        in_specs=[
            pl.BlockSpec((bs, 4 * D), lambda i: (i, 0)),
            pl.BlockSpec((4 * D, 2), lambda i: (0, 0)),
            pl.BlockSpec((1, 2), lambda i: (0, 0)),
        ],
        out_specs=pl.BlockSpec((bs, 2), lambda i: (i, 0)),
        out_shape=jax.ShapeDtypeStruct((B, 2), jnp.float32),
    )(acc, W, b.reshape(1, 2))


def kernel(x, table, W, b):
    B = x.shape[0]
    V = table.shape[0]
    x = _remap_idx(x.astype(jnp.int32))
    tbl_lin = _tc_retile(table.T)
    tbl2 = tbl_lin.reshape(tbl_lin.shape[0] // D, D)
    acc = _make_sc_pool(B)(x, tbl2)
    return _tc_tail(acc, W, b)
